# Initial kernel scaffold; baseline (speedup 1.0000x reference)
import jax
import jax.numpy as jnp
from jax.experimental import pallas as pl


def kernel(x_lane, x_agent, lane_batch, agent_batch, W_a, b_a, W1, b1, W2, b2, W_c):
    raise NotImplementedError("write your pallas kernel here")


# speedup vs baseline: 5.8923x; 5.8923x over previous
import functools

import jax
import jax.numpy as jnp
from jax import lax
from jax.experimental import pallas as pl
from jax.experimental.pallas import tpu as pltpu
from jax.experimental.pallas import tpu_sc as plsc

NSEG = 64
D = 128
CHUNK = 256


def _seg_accumulate(x_hbm, out_hbm, offs_v, acc_v, buf, seg, off_base, n_rows):
    ov = offs_v[pl.ds(off_base + seg, 16)]
    off0 = ov[0]
    off1 = ov[1]
    off0_al = (off0 // 8) * 8
    nch = (off1 - off0_al + (CHUNK - 1)) // CHUNK

    def chunk_body(i, accs):
        start = jnp.minimum(off0_al + i * CHUNK, n_rows - CHUNK)
        start = pl.multiple_of(start, 8)
        pltpu.sync_copy(x_hbm.at[pl.ds(start, CHUNK)], buf)

        def row_body(r, accs):
            g = start + r
            w = jnp.where((g >= off0) & (g < off1), 1.0, 0.0)
            return tuple(
                accs[j] + w * buf[r, pl.ds(j * 16, 16)] for j in range(8)
            )

        return lax.fori_loop(0, CHUNK, row_body, accs)

    zeros = tuple(jnp.zeros((16,), jnp.float32) for _ in range(8))
    accs = lax.fori_loop(0, nch, chunk_body, zeros)
    for j in range(8):
        acc_v[pl.ds(j * 16, 16)] = accs[j]
    pltpu.sync_copy(acc_v, out_hbm.at[seg])


def _make_sc_segsum(n_lane, n_agent):
    info = plsc.get_sparse_core_info()
    nc, ns = info.num_cores, info.num_subcores
    mesh = plsc.VectorSubcoreMesh(core_axis_name="c", subcore_axis_name="s")

    def body(xl_hbm, xa_hbm, offs_hbm, sum_l_hbm, sum_a_hbm,
             offs_v, acc_v, buf):
        wid = lax.axis_index("s") * nc + lax.axis_index("c")
        pltpu.sync_copy(offs_hbm, offs_v)
        for k in range(2):
            seg = wid * 2 + k
            _seg_accumulate(xl_hbm, sum_l_hbm, offs_v, acc_v, buf,
                            seg, 0, n_lane)
            _seg_accumulate(xa_hbm, sum_a_hbm, offs_v, acc_v, buf,
                            seg, 80, n_agent)

    return pl.kernel(
        body,
        out_type=(
            jax.ShapeDtypeStruct((NSEG, D), jnp.float32),
            jax.ShapeDtypeStruct((NSEG, D), jnp.float32),
        ),
        mesh=mesh,
        scratch_types=[
            pltpu.VMEM((160,), jnp.int32),
            pltpu.VMEM((D,), jnp.float32),
            pltpu.VMEM((CHUNK, D), jnp.float32),
        ],
    )


def _mlp_body(sl_ref, sa_ref, cl_ref, ca_ref, wa_ref, ba_ref,
              w1l_ref, w1a_ref, b1_ref, w2_ref, b2_ref, wc_ref, out_ref):
    cl = cl_ref[...]
    ca = ca_ref[...]
    ml = sl_ref[...] / jnp.maximum(cl, 1.0)
    ma = sa_ref[...] / jnp.maximum(ca, 1.0)
    up = jnp.dot(ma, wa_ref[...], preferred_element_type=jnp.float32)
    up = up + ba_ref[...]
    ctx_a = jnp.where(ca > 0.0, up, 0.0)
    h = (jnp.dot(ml, w1l_ref[...], preferred_element_type=jnp.float32)
         + jnp.dot(ctx_a, w1a_ref[...], preferred_element_type=jnp.float32)
         + b1_ref[...])
    h = h * jax.nn.sigmoid(h)
    ctx = jnp.dot(h, w2_ref[...], preferred_element_type=jnp.float32)
    ctx = ctx + b2_ref[...]
    out_ref[...] = jnp.dot(ctx, wc_ref[...], preferred_element_type=jnp.float32)


_mlp = pl.pallas_call(
    _mlp_body,
    out_shape=jax.ShapeDtypeStruct((NSEG, D), jnp.float32),
)


def kernel(x_lane, x_agent, lane_batch, agent_batch,
           W_a, b_a, W1, b1, W2, b2, W_c):
    n_lane = x_lane.shape[0]
    n_agent = x_agent.shape[0]
    segs = jnp.arange(NSEG, dtype=lane_batch.dtype)
    offs_l = jnp.searchsorted(lane_batch, segs, side="left").astype(jnp.int32)
    offs_l = jnp.concatenate(
        [offs_l, jnp.full((16,), n_lane, jnp.int32)])
    offs_a = jnp.searchsorted(agent_batch, segs, side="left").astype(jnp.int32)
    offs_a = jnp.concatenate(
        [offs_a, jnp.full((16,), n_agent, jnp.int32)])
    offs = jnp.concatenate([offs_l, offs_a])

    cnt_l = (offs_l[1:65] - offs_l[:64]).astype(jnp.float32)[:, None]
    cnt_a = (offs_a[1:65] - offs_a[:64]).astype(jnp.float32)[:, None]

    sum_l, sum_a = _make_sc_segsum(n_lane, n_agent)(x_lane, x_agent, offs)

    return _mlp(sum_l, sum_a, cnt_l, cnt_a,
                W_a.T, b_a[None, :],
                W1[:, :D].T, W1[:, D:].T, b1[None, :],
                W2.T, b2[None, :], W_c.T)
